# Initial kernel scaffold; baseline (speedup 1.0000x reference)
import jax
import jax.numpy as jnp
from jax.experimental import pallas as pl


def kernel(hidden_states, gumbel_u, W, b, codevectors):
    raise NotImplementedError("write your pallas kernel here")


# speedup vs baseline: 3.9421x; 3.9421x over previous
import functools

import jax
import jax.numpy as jnp
from jax.experimental import pallas as pl
from jax.experimental.pallas import tpu as pltpu

B, S, D_IN = 8, 512, 512
G, K, D_CODE = 2, 320, 256
DG = D_CODE // G
N = B * S
R = 512
GRID = N // R


def _tc_body(hs_ref, w_ref, b_ref, gu_ref, cv_ref, out_ref, ppl_ref, msum_ref):
    i = pl.program_id(0)
    logits = jnp.dot(hs_ref[...], w_ref[...], preferred_element_type=jnp.float32)
    logits = logits + b_ref[...]

    u = jnp.clip(gu_ref[...], 1e-10, 1.0 - 1e-10)
    z = logits - jnp.log(-jnp.log(u))

    iota = jax.lax.broadcasted_iota(jnp.int32, (R, K), 1)

    z0 = z[:, :K]
    m0 = jnp.max(z0, axis=1, keepdims=True)
    i0 = jnp.min(jnp.where(z0 >= m0, iota, K), axis=1, keepdims=True)
    oh0 = (iota == i0).astype(jnp.float32)
    z1 = z[:, K:]
    m1 = jnp.max(z1, axis=1, keepdims=True)
    i1 = jnp.min(jnp.where(z1 >= m1, iota, K), axis=1, keepdims=True)
    oh1 = (iota == i1).astype(jnp.float32)
    out0 = jnp.dot(oh0, cv_ref[:K, :], preferred_element_type=jnp.float32)
    out1 = jnp.dot(oh1, cv_ref[K:, :], preferred_element_type=jnp.float32)
    out_ref[...] = jnp.concatenate([out0, out1], axis=1)

    l0 = logits[:, :K]
    e0 = jnp.exp(l0 - jnp.max(l0, axis=1, keepdims=True))
    s0 = e0 / jnp.sum(e0, axis=1, keepdims=True)
    l1 = logits[:, K:]
    e1 = jnp.exp(l1 - jnp.max(l1, axis=1, keepdims=True))
    s1 = e1 / jnp.sum(e1, axis=1, keepdims=True)
    part = jnp.concatenate(
        [jnp.sum(s0, axis=0, keepdims=True), jnp.sum(s1, axis=0, keepdims=True)],
        axis=1,
    )

    @pl.when(i == 0)
    def _():
        msum_ref[...] = part

    @pl.when(i > 0)
    def _():
        msum_ref[...] += part

    @pl.when(i == GRID - 1)
    def _():
        m = msum_ref[...] / float(N)
        t = m * jnp.log(m + 1e-7)
        p0 = jnp.exp(-jnp.sum(t[:, :K], keepdims=True))
        p1 = jnp.exp(-jnp.sum(t[:, K:], keepdims=True))
        ppl_ref[...] = p0 + p1


@functools.partial(jax.jit)
def kernel(hidden_states, gumbel_u, W, b, codevectors):
    hs2 = hidden_states.reshape(N, D_IN)
    gu2 = gumbel_u.reshape(N, G * K)
    b2 = b.reshape(1, G * K)
    cv2 = codevectors.reshape(G * K, DG)

    out, ppl = pl.pallas_call(
        _tc_body,
        grid=(GRID,),
        in_specs=[
            pl.BlockSpec((R, D_IN), lambda i: (i, 0)),
            pl.BlockSpec((D_IN, G * K), lambda i: (0, 0)),
            pl.BlockSpec((1, G * K), lambda i: (0, 0)),
            pl.BlockSpec((R, G * K), lambda i: (i, 0)),
            pl.BlockSpec((G * K, DG), lambda i: (0, 0)),
        ],
        out_specs=[
            pl.BlockSpec((R, D_CODE), lambda i: (i, 0)),
            pl.BlockSpec((1, 1), lambda i: (0, 0)),
        ],
        out_shape=[
            jax.ShapeDtypeStruct((N, D_CODE), jnp.float32),
            jax.ShapeDtypeStruct((1, 1), jnp.float32),
        ],
        scratch_shapes=[pltpu.VMEM((1, G * K), jnp.float32)],
        compiler_params=pltpu.CompilerParams(
            dimension_semantics=("arbitrary",),
        ),
    )(hs2, W, b2, gu2, cv2)

    return out.reshape(B, S, D_CODE), ppl[0, 0]
